# baseline (speedup 1.0000x reference)
import functools

import jax
import jax.numpy as jnp
from jax import lax
from jax.experimental import pallas as pl
from jax.experimental.pallas import tpu as pltpu
from jax.experimental.pallas import tpu_sc as plsc

_LANES = 16
_NC = 2
_NS = 16
_C = 80
_R = 4


def _prep_call(x, ea, sfa, wcat, e_w, e_b, w_fr, b_fr, tf_W1, b2cat):
    N, D = x.shape
    E, DE = ea.shape
    H = D
    BE = 6400
    grid_n = E // BE
    BN = N // grid_n
    nt = wcat.shape[1] // D

    def body(x_ref, ea_ref, sfa_ref, wc_ref, we_ref, be_ref, wr_ref, br_ref,
             t1_ref, b2_ref, *outs):
        tabs = outs[:nt]
        off_ref, ofb_ref, ofr_ref, u_ref = outs[nt:]
        xb = x_ref[...]
        for t in range(nt):
            tabs[t][...] = jnp.dot(xb, wc_ref[:, t * D:(t + 1) * D],
                                   preferred_element_type=jnp.float32)
        e2 = jnp.dot(ea_ref[...], we_ref[...],
                     preferred_element_type=jnp.float32) + be_ref[...]
        off_ref[...] = e2[:, :H]
        ofb_ref[...] = e2[:, H:]
        ofr_ref[...] = jnp.dot(sfa_ref[...], wr_ref[...],
                               preferred_element_type=jnp.float32) + br_ref[...]

        @pl.when(pl.program_id(0) == 0)
        def _():
            for i in range(3):
                t1 = t1_ref[i * H:(i + 1) * H, :]
                u = jnp.dot(b2_ref[i:i + 1, :], t1,
                            preferred_element_type=jnp.float32)
                u_ref[i] = jnp.broadcast_to(u, (_C, H))

    full = lambda a: pl.BlockSpec(a.shape, lambda i: (0,) * a.ndim)
    return pl.pallas_call(
        body,
        grid=(grid_n,),
        in_specs=[pl.BlockSpec((BN, D), lambda i: (i, 0)),
                  pl.BlockSpec((BE, DE), lambda i: (i, 0)),
                  pl.BlockSpec((BE, DE), lambda i: (i, 0)),
                  full(wcat), full(e_w), full(e_b), full(w_fr), full(b_fr),
                  full(tf_W1), full(b2cat)],
        out_specs=[pl.BlockSpec((BN, D), lambda i: (i, 0))] * nt
        + [pl.BlockSpec((BE, H), lambda i: (i, 0))] * 3
        + [pl.BlockSpec((3, _C, H), lambda i: (0, 0, 0))],
        out_shape=[jax.ShapeDtypeStruct((N, D), jnp.float32)] * nt
        + [jax.ShapeDtypeStruct((E, H), jnp.float32)] * 3
        + [jax.ShapeDtypeStruct((3, _C, H), jnp.float32)],
    )(x, ea, sfa, wcat, e_w, e_b, w_fr, b_fr, tf_W1, b2cat)


def _sc_call(tables, eprojs, utile, future, past, early, later, n_pad):
    E = future.shape[0]
    H = tables[0].shape[1]
    per_tile = E // (_NC * _NS)
    steps = per_tile // _C
    zrows = n_pad // _NS
    zcopies = zrows // _C
    waves = (steps + 2 * _R - 1) // _R
    mesh = plsc.VectorSubcoreMesh(core_axis_name="c", subcore_axis_name="s")
    out_sds = jax.ShapeDtypeStruct((_NC, n_pad, H), jnp.float32)

    @functools.partial(
        pl.kernel,
        mesh=mesh,
        out_type=[out_sds] * 4,
        scratch_types=[
            pltpu.VMEM((_R, _C), jnp.int32),
            pltpu.VMEM((_R, _C), jnp.int32),
            pltpu.VMEM((_R, _C, H), jnp.float32),
            pltpu.SemaphoreType.DMA((_R,)),
            pltpu.SemaphoreType.DMA((_R,)),
            pltpu.SemaphoreType.DMA((_R,)),
            pltpu.SemaphoreType.DMA((_R,)),
            pltpu.VMEM_SHARED((n_pad, H), jnp.float32),
        ],
    )
    def k(tFf, tFp, tPp, tPf, tRe, tRl, eff, efb, efr, ut, fut, pas, ear, lat,
          oSff, oSfb, oSfr, oU,
          i1_v, i2_v, acc_v, semI, semE, semG, semS, s_sh):
        cid = lax.axis_index("c")
        sid = lax.axis_index("s")
        base_t = (cid * _NS + sid) * per_tile
        zeros16 = jnp.zeros((_LANES,), jnp.float32)

        def zero_acc():
            def zero_row(j, _):
                for g in range(H // _LANES):
                    acc_v[0, j, pl.ds(g * _LANES, _LANES)] = zeros16
                return 0
            lax.fori_loop(0, _C, zero_row, 0)
            for r in range(zcopies):
                pltpu.sync_copy(acc_v.at[0],
                                s_sh.at[pl.ds(sid * zrows + r * _C, _C)])
            plsc.subcore_barrier()

        def readback(out_hbm):
            plsc.subcore_barrier()
            pltpu.sync_copy(s_sh.at[pl.ds(sid * zrows, zrows)],
                            out_hbm.at[cid, pl.ds(sid * zrows, zrows)])

        def main_phase(tA, tB, ep, i1, i2, dual, out_hbm):
            zero_acc()

            def wait_scatter(s):
                pltpu.make_async_copy(acc_v.at[s], s_sh.at[i1_v.at[s]],
                                      semS.at[s]).wait()
                if dual:
                    pltpu.make_async_copy(acc_v.at[s], s_sh.at[i2_v.at[s]],
                                          semS.at[s]).wait()

            def wave(it, _):
                for r in range(_R):
                    p = it * _R + r

                    @pl.when(jnp.logical_and(p >= _R, p - _R < steps))
                    def _():
                        wait_scatter(r)

                    @pl.when(p < steps)
                    def _():
                        base = base_t + p * _C
                        pltpu.async_copy(i1.at[pl.ds(base, _C)], i1_v.at[r],
                                         semI.at[r])
                        pltpu.async_copy(i2.at[pl.ds(base, _C)], i2_v.at[r],
                                         semI.at[r])
                        pltpu.async_copy(ep.at[pl.ds(base, _C)], acc_v.at[r],
                                         semE.at[r])

                    @pl.when(jnp.logical_and(p >= 1, p - 1 < steps))
                    def _():
                        s = (r - 1) % _R
                        base = base_t + (p - 1) * _C
                        pltpu.make_async_copy(i1.at[pl.ds(base, _C)],
                                              i1_v.at[s], semI.at[s]).wait()
                        pltpu.make_async_copy(i2.at[pl.ds(base, _C)],
                                              i2_v.at[s], semI.at[s]).wait()
                        pltpu.make_async_copy(ep.at[pl.ds(base, _C)],
                                              acc_v.at[s], semE.at[s]).wait()
                        pltpu.async_copy(tA.at[i1_v.at[s]], acc_v.at[s],
                                         semG.at[s], add=True)
                        pltpu.async_copy(tB.at[i2_v.at[s]], acc_v.at[s],
                                         semG.at[s], add=True)

                    @pl.when(jnp.logical_and(p >= 2, p - 2 < steps))
                    def _():
                        s = (r - 2) % _R
                        pltpu.make_async_copy(tA.at[i1_v.at[s]], acc_v.at[s],
                                              semG.at[s]).wait()
                        pltpu.make_async_copy(tB.at[i2_v.at[s]], acc_v.at[s],
                                              semG.at[s]).wait()

                        def crow(j, _):
                            for g in range(H // _LANES):
                                sl = pl.ds(g * _LANES, _LANES)
                                acc_v[s, j, sl] = jnp.maximum(
                                    acc_v[s, j, sl], 0.0)
                            return 0
                        lax.fori_loop(0, _C, crow, 0)
                        pltpu.async_copy(acc_v.at[s], s_sh.at[i1_v.at[s]],
                                         semS.at[s], add=True)
                        if dual:
                            pltpu.async_copy(acc_v.at[s], s_sh.at[i2_v.at[s]],
                                             semS.at[s], add=True)
                return 0
            lax.fori_loop(0, waves, wave, 0)
            readback(out_hbm)

        main_phase(tFf, tFp, eff, fut, pas, False, oSff)
        main_phase(tPp, tPf, efb, pas, fut, False, oSfb)
        main_phase(tRe, tRl, efr, ear, lat, True, oSfr)

        zero_acc()
        for usel, idx in ((0, fut), (1, pas), (2, ear), (2, lat)):
            pltpu.sync_copy(ut.at[usel], acc_v.at[0])

            def bwave(it, _):
                for r in range(_R):
                    p = it * _R + r

                    @pl.when(jnp.logical_and(p >= _R, p - _R < steps))
                    def _():
                        pltpu.make_async_copy(acc_v.at[0],
                                              s_sh.at[i1_v.at[r]],
                                              semS.at[r]).wait()

                    @pl.when(p < steps)
                    def _():
                        pltpu.async_copy(idx.at[pl.ds(base_t + p * _C, _C)],
                                         i1_v.at[r], semI.at[r])

                    @pl.when(jnp.logical_and(p >= 1, p - 1 < steps))
                    def _():
                        s = (r - 1) % _R
                        pltpu.make_async_copy(
                            idx.at[pl.ds(base_t + (p - 1) * _C, _C)],
                            i1_v.at[s], semI.at[s]).wait()
                        pltpu.async_copy(acc_v.at[0], s_sh.at[i1_v.at[s]],
                                         semS.at[s], add=True)
                return 0
            lax.fori_loop(0, waves, bwave, 0)
        readback(oU)

    return k(*tables, *eprojs, utile, future, past, early, later)


def _final_call(S_ff, S_fr, S_fb, U, mcat, ff_W2, fr_W2, fb_W2,
                tfb1, tf_W2, tfb2, N):
    BN = 400
    H = tf_W2.shape[0]

    def body(sff, sfr, sfb, u_ref, m_ref, wf_ref, wr_ref, wb_ref,
             tb1_ref, t2_ref, tb2_ref, out_ref):
        def term(s_ref, row, w2_ref):
            s = s_ref[0] + s_ref[1]
            m = jnp.dot(w2_ref[...], m_ref[row * H:(row + 1) * H, :],
                        preferred_element_type=jnp.float32)
            return jnp.dot(s, m, preferred_element_type=jnp.float32)

        pre = (term(sff, 0, wf_ref) + term(sfr, 1, wr_ref)
               + term(sfb, 2, wb_ref) + u_ref[0] + u_ref[1] + tb1_ref[...])
        h = jnp.maximum(pre, 0.0)
        out_ref[...] = jnp.dot(h, t2_ref[...],
                               preferred_element_type=jnp.float32) + tb2_ref[...]

    sspec = pl.BlockSpec((_NC, BN, H), lambda i: (0, i, 0))
    full = lambda a: pl.BlockSpec(a.shape, lambda i: (0,) * a.ndim)
    return pl.pallas_call(
        body,
        grid=(N // BN,),
        in_specs=[sspec, sspec, sspec, sspec, full(mcat), full(ff_W2),
                  full(fr_W2), full(fb_W2), full(tfb1), full(tf_W2),
                  full(tfb2)],
        out_specs=pl.BlockSpec((BN, H), lambda i: (i, 0)),
        out_shape=jax.ShapeDtypeStruct((N, H), jnp.float32),
    )(S_ff, S_fr, S_fb, U, mcat, ff_W2, fr_W2, fb_W2, tfb1, tf_W2, tfb2)


def kernel(x, edge_index, edge_attr, same_frame_edge_index, same_frame_edge_attr,
           ff_W1, ff_b1, ff_W2, ff_b2,
           fr_W1, fr_b1, fr_W2, fr_b2,
           fb_W1, fb_b1, fb_W2, fb_b2,
           tf_W1, tf_b1, tf_W2, tf_b2):
    N, D = x.shape
    tile_rows = _NS * 640
    n_pad = -(-N // tile_rows) * tile_rows

    past, future = edge_index[0], edge_index[1]
    early, later = same_frame_edge_index[0], same_frame_edge_index[1]

    wcat = jnp.concatenate(
        [ff_W1[:D], ff_W1[D:2 * D],
         fb_W1[:D], fb_W1[D:2 * D],
         fr_W1[:D], fr_W1[D:2 * D]],
        axis=1)
    e_w = jnp.concatenate([ff_W1[2 * D:], fb_W1[2 * D:]], axis=1)
    e_b = jnp.concatenate([ff_b1, fb_b1])[None, :]
    b2cat = jnp.stack([ff_b2, fr_b2, fb_b2])

    (P_ff_f, P_ff_p, P_fb_p, P_fb_f, P_fr_e, P_fr_l,
     e_ff, e_fb, e_fr, utile) = _prep_call(
        x, edge_attr, same_frame_edge_attr, wcat, e_w, e_b,
        fr_W1[2 * D:], fr_b1[None, :], tf_W1, b2cat)

    S_ff, S_fb, S_fr, U = _sc_call(
        (P_ff_f, P_ff_p, P_fb_p, P_fb_f, P_fr_e, P_fr_l),
        (e_ff, e_fb, e_fr), utile, future, past, early, later, n_pad)

    return _final_call(S_ff, S_fr, S_fb, U, tf_W1, ff_W2, fr_W2, fb_W2,
                       tf_b1[None, :], tf_W2, tf_b2[None, :], N)


# speedup vs baseline: 1.0192x; 1.0192x over previous
import functools

import jax
import jax.numpy as jnp
from jax import lax
from jax.experimental import pallas as pl
from jax.experimental.pallas import tpu as pltpu
from jax.experimental.pallas import tpu_sc as plsc

_LANES = 16
_NC = 2
_NS = 16
_C = 80
_R = 3


def _prep_call(x, ea, sfa, wcat, e_w, e_b, w_fr, b_fr, tf_W1, b2cat):
    N, D = x.shape
    E, DE = ea.shape
    H = D
    BE = 6400
    grid_n = E // BE
    BN = N // grid_n
    nt = wcat.shape[1] // D

    def body(x_ref, ea_ref, sfa_ref, wc_ref, we_ref, be_ref, wr_ref, br_ref,
             t1_ref, b2_ref, *outs):
        tabs = outs[:nt]
        off_ref, ofb_ref, ofr_ref, u_ref = outs[nt:]
        xb = x_ref[...]
        for t in range(nt):
            tabs[t][...] = jnp.dot(xb, wc_ref[:, t * D:(t + 1) * D],
                                   preferred_element_type=jnp.float32)
        e2 = jnp.dot(ea_ref[...], we_ref[...],
                     preferred_element_type=jnp.float32) + be_ref[...]
        off_ref[...] = e2[:, :H]
        ofb_ref[...] = e2[:, H:]
        ofr_ref[...] = jnp.dot(sfa_ref[...], wr_ref[...],
                               preferred_element_type=jnp.float32) + br_ref[...]

        @pl.when(pl.program_id(0) == 0)
        def _():
            for i in range(3):
                t1 = t1_ref[i * H:(i + 1) * H, :]
                u = jnp.dot(b2_ref[i:i + 1, :], t1,
                            preferred_element_type=jnp.float32)
                u_ref[i] = jnp.broadcast_to(u, (_C, H))

    full = lambda a: pl.BlockSpec(a.shape, lambda i: (0,) * a.ndim)
    return pl.pallas_call(
        body,
        grid=(grid_n,),
        in_specs=[pl.BlockSpec((BN, D), lambda i: (i, 0)),
                  pl.BlockSpec((BE, DE), lambda i: (i, 0)),
                  pl.BlockSpec((BE, DE), lambda i: (i, 0)),
                  full(wcat), full(e_w), full(e_b), full(w_fr), full(b_fr),
                  full(tf_W1), full(b2cat)],
        out_specs=[pl.BlockSpec((BN, D), lambda i: (i, 0))] * nt
        + [pl.BlockSpec((BE, H), lambda i: (i, 0))] * 3
        + [pl.BlockSpec((3, _C, H), lambda i: (0, 0, 0))],
        out_shape=[jax.ShapeDtypeStruct((N, D), jnp.float32)] * nt
        + [jax.ShapeDtypeStruct((E, H), jnp.float32)] * 3
        + [jax.ShapeDtypeStruct((3, _C, H), jnp.float32)],
    )(x, ea, sfa, wcat, e_w, e_b, w_fr, b_fr, tf_W1, b2cat)


def _sc_call(tables, eprojs, utile, future, past, early, later, n_pad):
    E = future.shape[0]
    H = tables[0].shape[1]
    per_tile = E // (_NC * _NS)
    steps = per_tile // _C
    zrows = n_pad // _NS
    zcopies = zrows // _C
    waves = (steps + 2 * _R - 1) // _R
    mesh = plsc.VectorSubcoreMesh(core_axis_name="c", subcore_axis_name="s")
    out_sds = jax.ShapeDtypeStruct((_NC, n_pad, H), jnp.float32)

    @functools.partial(
        pl.kernel,
        mesh=mesh,
        out_type=[out_sds] * 4,
        scratch_types=[
            pltpu.VMEM((_R, _C), jnp.int32),
            pltpu.VMEM((_R, _C), jnp.int32),
            pltpu.VMEM((_R, _C, H), jnp.float32),
            pltpu.SemaphoreType.DMA((_R,)),
            pltpu.SemaphoreType.DMA((_R,)),
            pltpu.SemaphoreType.DMA((_R,)),
            pltpu.SemaphoreType.DMA((_R,)),
            pltpu.VMEM_SHARED((n_pad, H), jnp.float32),
        ],
    )
    def k(tFf, tFp, tPp, tPf, tRe, tRl, eff, efb, efr, ut, fut, pas, ear, lat,
          oSff, oSfb, oSfr, oU,
          i1_v, i2_v, acc_v, semI, semE, semG, semS, s_sh):
        cid = lax.axis_index("c")
        sid = lax.axis_index("s")
        base_t = (cid * _NS + sid) * per_tile
        zeros16 = jnp.zeros((_LANES,), jnp.float32)

        def zero_acc():
            def zero_row(j, _):
                for g in range(H // _LANES):
                    acc_v[0, j, pl.ds(g * _LANES, _LANES)] = zeros16
                return 0
            lax.fori_loop(0, _C, zero_row, 0)
            for r in range(zcopies):
                pltpu.sync_copy(acc_v.at[0],
                                s_sh.at[pl.ds(sid * zrows + r * _C, _C)])
            plsc.subcore_barrier()

        def readback(out_hbm):
            plsc.subcore_barrier()
            pltpu.sync_copy(s_sh.at[pl.ds(sid * zrows, zrows)],
                            out_hbm.at[cid, pl.ds(sid * zrows, zrows)])

        def main_phase(tA, tB, ep, i1, i2, dual, out_hbm):
            zero_acc()

            def wait_scatter(s):
                pltpu.make_async_copy(acc_v.at[s], s_sh.at[i1_v.at[s]],
                                      semS.at[s]).wait()
                if dual:
                    pltpu.make_async_copy(acc_v.at[s], s_sh.at[i2_v.at[s]],
                                          semS.at[s]).wait()

            def wave(it, _):
                for r in range(_R):
                    p = it * _R + r

                    @pl.when(jnp.logical_and(p >= _R, p - _R < steps))
                    def _():
                        wait_scatter(r)

                    @pl.when(p < steps)
                    def _():
                        base = base_t + p * _C
                        pltpu.async_copy(i1.at[pl.ds(base, _C)], i1_v.at[r],
                                         semI.at[r])
                        pltpu.async_copy(i2.at[pl.ds(base, _C)], i2_v.at[r],
                                         semI.at[r])
                        pltpu.async_copy(ep.at[pl.ds(base, _C)], acc_v.at[r],
                                         semE.at[r])

                    @pl.when(jnp.logical_and(p >= 1, p - 1 < steps))
                    def _():
                        s = (r - 1) % _R
                        base = base_t + (p - 1) * _C
                        pltpu.make_async_copy(i1.at[pl.ds(base, _C)],
                                              i1_v.at[s], semI.at[s]).wait()
                        pltpu.make_async_copy(i2.at[pl.ds(base, _C)],
                                              i2_v.at[s], semI.at[s]).wait()
                        pltpu.make_async_copy(ep.at[pl.ds(base, _C)],
                                              acc_v.at[s], semE.at[s]).wait()
                        pltpu.async_copy(tA.at[i1_v.at[s]], acc_v.at[s],
                                         semG.at[s], add=True)
                        pltpu.async_copy(tB.at[i2_v.at[s]], acc_v.at[s],
                                         semG.at[s], add=True)

                    @pl.when(jnp.logical_and(p >= 2, p - 2 < steps))
                    def _():
                        s = (r - 2) % _R
                        pltpu.make_async_copy(tA.at[i1_v.at[s]], acc_v.at[s],
                                              semG.at[s]).wait()
                        pltpu.make_async_copy(tB.at[i2_v.at[s]], acc_v.at[s],
                                              semG.at[s]).wait()

                        def crow(j, _):
                            for g in range(H // _LANES):
                                sl = pl.ds(g * _LANES, _LANES)
                                acc_v[s, j, sl] = jnp.maximum(
                                    acc_v[s, j, sl], 0.0)
                            return 0
                        lax.fori_loop(0, _C, crow, 0)
                        pltpu.async_copy(acc_v.at[s], s_sh.at[i1_v.at[s]],
                                         semS.at[s], add=True)
                        if dual:
                            pltpu.async_copy(acc_v.at[s], s_sh.at[i2_v.at[s]],
                                             semS.at[s], add=True)
                return 0
            lax.fori_loop(0, waves, wave, 0)
            readback(out_hbm)

        main_phase(tFf, tFp, eff, fut, pas, False, oSff)
        main_phase(tPp, tPf, efb, pas, fut, False, oSfb)
        main_phase(tRe, tRl, efr, ear, lat, True, oSfr)

        zero_acc()
        for usel, idx in ((0, fut), (1, pas), (2, ear), (2, lat)):
            pltpu.sync_copy(ut.at[usel], acc_v.at[0])

            def bwave(it, _):
                for r in range(_R):
                    p = it * _R + r

                    @pl.when(jnp.logical_and(p >= _R, p - _R < steps))
                    def _():
                        pltpu.make_async_copy(acc_v.at[0],
                                              s_sh.at[i1_v.at[r]],
                                              semS.at[r]).wait()

                    @pl.when(p < steps)
                    def _():
                        pltpu.async_copy(idx.at[pl.ds(base_t + p * _C, _C)],
                                         i1_v.at[r], semI.at[r])

                    @pl.when(jnp.logical_and(p >= 1, p - 1 < steps))
                    def _():
                        s = (r - 1) % _R
                        pltpu.make_async_copy(
                            idx.at[pl.ds(base_t + (p - 1) * _C, _C)],
                            i1_v.at[s], semI.at[s]).wait()
                        pltpu.async_copy(acc_v.at[0], s_sh.at[i1_v.at[s]],
                                         semS.at[s], add=True)
                return 0
            lax.fori_loop(0, waves, bwave, 0)
        readback(oU)

    return k(*tables, *eprojs, utile, future, past, early, later)


def _final_call(S_ff, S_fr, S_fb, U, mcat, ff_W2, fr_W2, fb_W2,
                tfb1, tf_W2, tfb2, N):
    BN = 400
    H = tf_W2.shape[0]

    def body(sff, sfr, sfb, u_ref, m_ref, wf_ref, wr_ref, wb_ref,
             tb1_ref, t2_ref, tb2_ref, out_ref):
        def term(s_ref, row, w2_ref):
            s = s_ref[0] + s_ref[1]
            m = jnp.dot(w2_ref[...], m_ref[row * H:(row + 1) * H, :],
                        preferred_element_type=jnp.float32)
            return jnp.dot(s, m, preferred_element_type=jnp.float32)

        pre = (term(sff, 0, wf_ref) + term(sfr, 1, wr_ref)
               + term(sfb, 2, wb_ref) + u_ref[0] + u_ref[1] + tb1_ref[...])
        h = jnp.maximum(pre, 0.0)
        out_ref[...] = jnp.dot(h, t2_ref[...],
                               preferred_element_type=jnp.float32) + tb2_ref[...]

    sspec = pl.BlockSpec((_NC, BN, H), lambda i: (0, i, 0))
    full = lambda a: pl.BlockSpec(a.shape, lambda i: (0,) * a.ndim)
    return pl.pallas_call(
        body,
        grid=(N // BN,),
        in_specs=[sspec, sspec, sspec, sspec, full(mcat), full(ff_W2),
                  full(fr_W2), full(fb_W2), full(tfb1), full(tf_W2),
                  full(tfb2)],
        out_specs=pl.BlockSpec((BN, H), lambda i: (i, 0)),
        out_shape=jax.ShapeDtypeStruct((N, H), jnp.float32),
    )(S_ff, S_fr, S_fb, U, mcat, ff_W2, fr_W2, fb_W2, tfb1, tf_W2, tfb2)


def kernel(x, edge_index, edge_attr, same_frame_edge_index, same_frame_edge_attr,
           ff_W1, ff_b1, ff_W2, ff_b2,
           fr_W1, fr_b1, fr_W2, fr_b2,
           fb_W1, fb_b1, fb_W2, fb_b2,
           tf_W1, tf_b1, tf_W2, tf_b2):
    N, D = x.shape
    tile_rows = _NS * 640
    n_pad = -(-N // tile_rows) * tile_rows

    past, future = edge_index[0], edge_index[1]
    early, later = same_frame_edge_index[0], same_frame_edge_index[1]

    wcat = jnp.concatenate(
        [ff_W1[:D], ff_W1[D:2 * D],
         fb_W1[:D], fb_W1[D:2 * D],
         fr_W1[:D], fr_W1[D:2 * D]],
        axis=1)
    e_w = jnp.concatenate([ff_W1[2 * D:], fb_W1[2 * D:]], axis=1)
    e_b = jnp.concatenate([ff_b1, fb_b1])[None, :]
    b2cat = jnp.stack([ff_b2, fr_b2, fb_b2])

    (P_ff_f, P_ff_p, P_fb_p, P_fb_f, P_fr_e, P_fr_l,
     e_ff, e_fb, e_fr, utile) = _prep_call(
        x, edge_attr, same_frame_edge_attr, wcat, e_w, e_b,
        fr_W1[2 * D:], fr_b1[None, :], tf_W1, b2cat)

    S_ff, S_fb, S_fr, U = _sc_call(
        (P_ff_f, P_ff_p, P_fb_p, P_fb_f, P_fr_e, P_fr_l),
        (e_ff, e_fb, e_fr), utile, future, past, early, later, n_pad)

    return _final_call(S_ff, S_fr, S_fb, U, tf_W1, ff_W2, fr_W2, fb_W2,
                       tf_b1[None, :], tf_W2, tf_b2[None, :], N)
